# baseline (speedup 1.0000x reference)
import functools

import jax
import jax.numpy as jnp
from jax import lax
from jax.experimental import pallas as pl
from jax.experimental.pallas import tpu as pltpu
from jax.experimental.pallas import tpu_sc as plsc

_NK = 8
_K = _NK + 1
_NCAND = 10
_N = 16384
_B = 32
_L = 16
_U = 16
_MASKED = 1e18


def _f32(x):
    return jnp.float32(x)


def _sc_kernel(kpts_hbm, cpt_hbm, pcld_hbm, seg_hbm, out_hbm,
               xb, yb, zb, mk, best_v, best_i, idx48, g48, p48, outb, sem):
    b = lax.axis_index("s") * 2 + lax.axis_index("c")

    lane = lax.broadcasted_iota(jnp.int32, (_L,), 0)
    inf_v = jnp.full((_L,), jnp.inf, jnp.float32)
    zero_v = jnp.full((_L,), _f32(0.0))
    big_v = jnp.full((_L,), _f32(_MASKED))

    n_groups = _N // _L

    pltpu.sync_copy(seg_hbm.at[pl.ds((b * 2 + 0) * _N, _N)], xb)
    pltpu.sync_copy(seg_hbm.at[pl.ds((b * 2 + 1) * _N, _N)], yb)

    def mask_body(i, carry):
        for j in range(8):
            sl = pl.ds((i * 8 + j) * _L, _L)
            mk[sl] = jnp.where(yb[sl] > xb[sl], zero_v, big_v)
        return carry

    lax.fori_loop(0, n_groups // 8, mask_body, jnp.int32(0))

    for k in range(_K):
        if k < _NK:
            pb = (b * 24 + 3 * k) * _N
            src = kpts_hbm
        else:
            pb = (b * 3) * _N
            src = cpt_hbm
        pltpu.sync_copy(src.at[pl.ds(pb, _N)], xb)
        pltpu.sync_copy(src.at[pl.ds(pb + _N, _N)], yb)
        pltpu.sync_copy(src.at[pl.ds(pb + 2 * _N, _N)], zb)

        best_v[pl.ds(k * _L, _L)] = inf_v
        best_i[pl.ds(k * _L, _L)] = lane * 0

        def do_merge(mn, g, thr, k=k):
            def merge(mn=mn, g=g, k=k):
                gidx = g * _L + lane
                sv, si = plsc.sort_key_val(mn, gidx)
                rv = lax.rev(sv, (0,))
                ri = lax.rev(si, (0,))
                bv = best_v[pl.ds(k * _L, _L)]
                bi = best_i[pl.ds(k * _L, _L)]
                keep_old = bv <= rv
                mv = jnp.where(keep_old, bv, rv)
                mi = jnp.where(keep_old, bi, ri)
                mv, mi = plsc.sort_key_val(mv, mi)
                best_v[pl.ds(k * _L, _L)] = mv
                best_i[pl.ds(k * _L, _L)] = mi
                return jnp.min(jnp.where(lane == _NCAND - 1, mv, inf_v))

            def keep():
                return thr

            return lax.cond(jnp.min(mn) < thr, merge, keep)

        def scan_body(i, thr, k=k):
            g0 = i * _U
            mns = []
            for j in range(_U):
                sl = pl.ds((i * _U + j) * _L, _L)
                x = xb[sl]
                y = yb[sl]
                z = zb[sl]
                mns.append(x * x + y * y + z * z + mk[sl])
            t = mns
            while len(t) > 1:
                t = [jnp.minimum(t[2 * a], t[2 * a + 1])
                     for a in range(len(t) // 2)]
            blockmin = jnp.min(t[0])

            def hit(thr=thr, mns=mns, g0=g0, k=k):
                tc = thr
                for j in range(_U):
                    tc = do_merge(mns[j], g0 + j, tc, k=k)
                return tc

            def miss():
                return thr

            return lax.cond(blockmin < thr, hit, miss)

        lax.fori_loop(0, n_groups // _U, scan_body, _f32(jnp.inf))

    imax = jnp.full((_L,), jnp.int32(2147483647), jnp.int32)
    for k in range(_K):
        bv = best_v[pl.ds(k * _L, _L)]
        bi = best_i[pl.ds(k * _L, _L)]
        done = lane < 0
        for _ in range(_NCAND):
            cur = jnp.where(done, inf_v, bv)
            m = jnp.min(cur)
            candm = cur == jnp.full((_L,), _f32(1.0)) * m
            mi = jnp.min(jnp.where(candm, bi, imax))
            done = done | (candm & (bi == mi))

        if k < _NK:
            gbase = (b * 24 + 3 * k) * _N
            gsrc = kpts_hbm
        else:
            gbase = (b * 3) * _N
            gsrc = cpt_hbm
        idx48[pl.ds(0, _L)] = gbase + bi
        idx48[pl.ds(_L, _L)] = gbase + _N + bi
        idx48[pl.ds(2 * _L, _L)] = gbase + 2 * _N + bi
        pltpu.async_copy(gsrc.at[idx48], g48, sem).wait()
        pbase = (b * 3) * _N
        idx48[pl.ds(0, _L)] = pbase + bi
        idx48[pl.ds(_L, _L)] = pbase + _N + bi
        idx48[pl.ds(2 * _L, _L)] = pbase + 2 * _N + bi
        pltpu.async_copy(pcld_hbm.at[idx48], p48, sem).wait()
        cx = p48[pl.ds(0, _L)] + g48[pl.ds(0, _L)]
        cy = p48[pl.ds(_L, _L)] + g48[pl.ds(_L, _L)]
        cz = p48[pl.ds(2 * _L, _L)] + g48[pl.ds(2 * _L, _L)]

        w10 = jnp.where(done, jnp.full((_L,), _f32(1.0)), zero_v)
        inv10 = _f32(0.1)
        mx = jnp.sum(w10 * cx) * inv10
        my = jnp.sum(w10 * cy) * inv10
        mz = jnp.sum(w10 * cz) * inv10
        dx = cx - mx
        dy = cy - my
        dz = cz - mz
        vx = jnp.sum(w10 * dx * dx) * inv10
        vy = jnp.sum(w10 * dy * dy) * inv10
        vz = jnp.sum(w10 * dz * dz) * inv10

        def _thr2(var):
            varv = jnp.full((_L,), _f32(1.0)) * var
            i = lax.bitcast_convert_type(jnp.maximum(varv, _f32(1e-30)),
                                         jnp.int32)
            s_a = lax.bitcast_convert_type(
                (i >> 1) + jnp.int32(0x1FBD1DF6), jnp.float32)
            s_a = _f32(0.5) * (s_a + varv / s_a)
            s_a = _f32(0.5) * (s_a + varv / s_a)
            s_a = jnp.where(varv > zero_v, s_a, zero_v)
            return varv + _f32(2e-9) * s_a + _f32(1e-18)

        inl = ((dx * dx <= _thr2(vx))
               & (dy * dy <= _thr2(vy))
               & (dz * dz <= _thr2(vz))
               & done)
        ones = jnp.full((_L,), _f32(1.0))
        w = jnp.where(inl, ones, zero_v)
        denv = ones * (jnp.sum(w) + _f32(1e-8))
        oxv = (ones * jnp.sum(w * cx)) / denv
        oyv = (ones * jnp.sum(w * cy)) / denv
        ozv = (ones * jnp.sum(w * cz)) / denv

        outv = jnp.where(lane == 0, oxv,
                         jnp.where(lane == 1, oyv,
                                   jnp.where(lane == 2, ozv, zero_v)))
        outb[pl.ds(k * _L, _L)] = outv

    pltpu.sync_copy(outb, out_hbm.at[pl.ds(b * _K * _L, _K * _L)])


def kernel(pcld_input, kpts_pre_input, cpt_pre_input, seg_pre_input):
    b, n, nk, _ = kpts_pre_input.shape
    assert (b, n, nk) == (_B, _N, _NK)
    kpts_t = jnp.transpose(kpts_pre_input.reshape(b, n, 3 * _NK),
                           (0, 2, 1)).reshape(-1)
    cpt_t = jnp.transpose(cpt_pre_input.reshape(b, n, 3), (0, 2, 1)
                          ).reshape(-1)
    pcld_t = jnp.transpose(pcld_input, (0, 2, 1)).reshape(-1)
    seg_t = jnp.transpose(seg_pre_input, (0, 2, 1)).reshape(-1)

    mesh = plsc.VectorSubcoreMesh(core_axis_name="c", subcore_axis_name="s")
    run = functools.partial(
        pl.kernel,
        mesh=mesh,
        compiler_params=pltpu.CompilerParams(needs_layout_passes=False),
        out_type=jax.ShapeDtypeStruct((b * _K * _L,), jnp.float32),
        scratch_types=[
            pltpu.VMEM((_N,), jnp.float32),
            pltpu.VMEM((_N,), jnp.float32),
            pltpu.VMEM((_N,), jnp.float32),
            pltpu.VMEM((_N,), jnp.float32),
            pltpu.VMEM((_K * _L,), jnp.float32),
            pltpu.VMEM((_K * _L,), jnp.int32),
            pltpu.VMEM((3 * _L,), jnp.int32),
            pltpu.VMEM((3 * _L,), jnp.float32),
            pltpu.VMEM((3 * _L,), jnp.float32),
            pltpu.VMEM((_K * _L,), jnp.float32),
            pltpu.SemaphoreType.DMA,
        ],
    )(_sc_kernel)
    out16 = run(kpts_t, cpt_t, pcld_t, seg_t)
    return out16.reshape(b, _K, _L)[:, :, :3]


# speedup vs baseline: 1.0913x; 1.0913x over previous
import functools

import jax
import jax.numpy as jnp
from jax import lax
from jax.experimental import pallas as pl
from jax.experimental.pallas import tpu as pltpu
from jax.experimental.pallas import tpu_sc as plsc

_NK = 8
_K = _NK + 1
_NCAND = 10
_N = 16384
_B = 32
_L = 16
_U = 8
_MASKED = 1e18


def _f32(x):
    return jnp.float32(x)


def _sc_kernel(kpts_hbm, cpt_hbm, pcld_hbm, seg_hbm, out_hbm,
               xb, yb, zb, mk, best_v, best_i, idx48, g48, p48, outb, sem):
    b = lax.axis_index("s") * 2 + lax.axis_index("c")

    lane = lax.broadcasted_iota(jnp.int32, (_L,), 0)
    inf_v = jnp.full((_L,), jnp.inf, jnp.float32)
    zero_v = jnp.full((_L,), _f32(0.0))
    big_v = jnp.full((_L,), _f32(_MASKED))

    n_groups = _N // _L

    pltpu.sync_copy(seg_hbm.at[pl.ds((b * 2 + 0) * _N, _N)], xb)
    pltpu.sync_copy(seg_hbm.at[pl.ds((b * 2 + 1) * _N, _N)], yb)

    def mask_body(i, carry):
        for j in range(8):
            sl = pl.ds((i * 8 + j) * _L, _L)
            mk[sl] = jnp.where(yb[sl] > xb[sl], zero_v, big_v)
        return carry

    lax.fori_loop(0, n_groups // 8, mask_body, jnp.int32(0))

    for k in range(_K):
        if k < _NK:
            pb = (b * 24 + 3 * k) * _N
            src = kpts_hbm
        else:
            pb = (b * 3) * _N
            src = cpt_hbm
        pltpu.sync_copy(src.at[pl.ds(pb, _N)], xb)
        pltpu.sync_copy(src.at[pl.ds(pb + _N, _N)], yb)
        pltpu.sync_copy(src.at[pl.ds(pb + 2 * _N, _N)], zb)

        best_v[pl.ds(k * _L, _L)] = inf_v
        best_i[pl.ds(k * _L, _L)] = lane * 0

        def do_merge(mn, g, thr, k=k):
            def merge(mn=mn, g=g, k=k):
                gidx = g * _L + lane
                sv, si = plsc.sort_key_val(mn, gidx)
                rv = lax.rev(sv, (0,))
                ri = lax.rev(si, (0,))
                bv = best_v[pl.ds(k * _L, _L)]
                bi = best_i[pl.ds(k * _L, _L)]
                keep_old = bv <= rv
                mv = jnp.where(keep_old, bv, rv)
                mi = jnp.where(keep_old, bi, ri)
                mv, mi = plsc.sort_key_val(mv, mi)
                best_v[pl.ds(k * _L, _L)] = mv
                best_i[pl.ds(k * _L, _L)] = mi
                return jnp.min(jnp.where(lane == _NCAND - 1, mv, inf_v))

            def keep():
                return thr

            return lax.cond(jnp.min(mn) < thr, merge, keep)

        def scan_body(i, thr, k=k):
            g0 = i * _U
            mns = []
            for j in range(_U):
                sl = pl.ds((i * _U + j) * _L, _L)
                x = xb[sl]
                y = yb[sl]
                z = zb[sl]
                mns.append(x * x + y * y + z * z + mk[sl])
            t = mns
            while len(t) > 1:
                t = [jnp.minimum(t[2 * a], t[2 * a + 1])
                     for a in range(len(t) // 2)]
            blockmin = jnp.min(t[0])

            def hit(thr=thr, mns=mns, g0=g0, k=k):
                tc = thr
                for j in range(_U):
                    tc = do_merge(mns[j], g0 + j, tc, k=k)
                return tc

            def miss():
                return thr

            return lax.cond(blockmin < thr, hit, miss)

        lax.fori_loop(0, n_groups // _U, scan_body, _f32(jnp.inf))

    imax = jnp.full((_L,), jnp.int32(2147483647), jnp.int32)
    for k in range(_K):
        bv = best_v[pl.ds(k * _L, _L)]
        bi = best_i[pl.ds(k * _L, _L)]
        done = lane < 0
        for _ in range(_NCAND):
            cur = jnp.where(done, inf_v, bv)
            m = jnp.min(cur)
            candm = cur == jnp.full((_L,), _f32(1.0)) * m
            mi = jnp.min(jnp.where(candm, bi, imax))
            done = done | (candm & (bi == mi))

        if k < _NK:
            gbase = (b * 24 + 3 * k) * _N
            gsrc = kpts_hbm
        else:
            gbase = (b * 3) * _N
            gsrc = cpt_hbm
        idx48[pl.ds(0, _L)] = gbase + bi
        idx48[pl.ds(_L, _L)] = gbase + _N + bi
        idx48[pl.ds(2 * _L, _L)] = gbase + 2 * _N + bi
        pltpu.async_copy(gsrc.at[idx48], g48, sem).wait()
        pbase = (b * 3) * _N
        idx48[pl.ds(0, _L)] = pbase + bi
        idx48[pl.ds(_L, _L)] = pbase + _N + bi
        idx48[pl.ds(2 * _L, _L)] = pbase + 2 * _N + bi
        pltpu.async_copy(pcld_hbm.at[idx48], p48, sem).wait()
        cx = p48[pl.ds(0, _L)] + g48[pl.ds(0, _L)]
        cy = p48[pl.ds(_L, _L)] + g48[pl.ds(_L, _L)]
        cz = p48[pl.ds(2 * _L, _L)] + g48[pl.ds(2 * _L, _L)]

        w10 = jnp.where(done, jnp.full((_L,), _f32(1.0)), zero_v)
        inv10 = _f32(0.1)
        mx = jnp.sum(w10 * cx) * inv10
        my = jnp.sum(w10 * cy) * inv10
        mz = jnp.sum(w10 * cz) * inv10
        dx = cx - mx
        dy = cy - my
        dz = cz - mz
        vx = jnp.sum(w10 * dx * dx) * inv10
        vy = jnp.sum(w10 * dy * dy) * inv10
        vz = jnp.sum(w10 * dz * dz) * inv10

        def _thr2(var):
            varv = jnp.full((_L,), _f32(1.0)) * var
            i = lax.bitcast_convert_type(jnp.maximum(varv, _f32(1e-30)),
                                         jnp.int32)
            s_a = lax.bitcast_convert_type(
                (i >> 1) + jnp.int32(0x1FBD1DF6), jnp.float32)
            s_a = _f32(0.5) * (s_a + varv / s_a)
            s_a = _f32(0.5) * (s_a + varv / s_a)
            s_a = jnp.where(varv > zero_v, s_a, zero_v)
            return varv + _f32(2e-9) * s_a + _f32(1e-18)

        inl = ((dx * dx <= _thr2(vx))
               & (dy * dy <= _thr2(vy))
               & (dz * dz <= _thr2(vz))
               & done)
        ones = jnp.full((_L,), _f32(1.0))
        w = jnp.where(inl, ones, zero_v)
        denv = ones * (jnp.sum(w) + _f32(1e-8))
        oxv = (ones * jnp.sum(w * cx)) / denv
        oyv = (ones * jnp.sum(w * cy)) / denv
        ozv = (ones * jnp.sum(w * cz)) / denv

        outv = jnp.where(lane == 0, oxv,
                         jnp.where(lane == 1, oyv,
                                   jnp.where(lane == 2, ozv, zero_v)))
        outb[pl.ds(k * _L, _L)] = outv

    pltpu.sync_copy(outb, out_hbm.at[pl.ds(b * _K * _L, _K * _L)])


def kernel(pcld_input, kpts_pre_input, cpt_pre_input, seg_pre_input):
    b, n, nk, _ = kpts_pre_input.shape
    assert (b, n, nk) == (_B, _N, _NK)
    kpts_t = jnp.transpose(kpts_pre_input.reshape(b, n, 3 * _NK),
                           (0, 2, 1)).reshape(-1)
    cpt_t = jnp.transpose(cpt_pre_input.reshape(b, n, 3), (0, 2, 1)
                          ).reshape(-1)
    pcld_t = jnp.transpose(pcld_input, (0, 2, 1)).reshape(-1)
    seg_t = jnp.transpose(seg_pre_input, (0, 2, 1)).reshape(-1)

    mesh = plsc.VectorSubcoreMesh(core_axis_name="c", subcore_axis_name="s")
    run = functools.partial(
        pl.kernel,
        mesh=mesh,
        compiler_params=pltpu.CompilerParams(needs_layout_passes=False),
        out_type=jax.ShapeDtypeStruct((b * _K * _L,), jnp.float32),
        scratch_types=[
            pltpu.VMEM((_N,), jnp.float32),
            pltpu.VMEM((_N,), jnp.float32),
            pltpu.VMEM((_N,), jnp.float32),
            pltpu.VMEM((_N,), jnp.float32),
            pltpu.VMEM((_K * _L,), jnp.float32),
            pltpu.VMEM((_K * _L,), jnp.int32),
            pltpu.VMEM((3 * _L,), jnp.int32),
            pltpu.VMEM((3 * _L,), jnp.float32),
            pltpu.VMEM((3 * _L,), jnp.float32),
            pltpu.VMEM((_K * _L,), jnp.float32),
            pltpu.SemaphoreType.DMA,
        ],
    )(_sc_kernel)
    out16 = run(kpts_t, cpt_t, pcld_t, seg_t)
    return out16.reshape(b, _K, _L)[:, :, :3]


# speedup vs baseline: 1.1865x; 1.0873x over previous
import functools

import jax
import jax.numpy as jnp
from jax import lax
from jax.experimental import pallas as pl
from jax.experimental.pallas import tpu as pltpu
from jax.experimental.pallas import tpu_sc as plsc

_NK = 8
_K = _NK + 1
_NCAND = 10
_N = 16384
_B = 32
_L = 16
_U = 8
_MASKED = 1e18


def _f32(x):
    return jnp.float32(x)


def _sc_kernel(kpts_hbm, cpt_hbm, pcld_hbm, seg_hbm, out_hbm,
               xb, yb, zb, xb2, yb2, zb2, mk, best_v, best_i, idx48, g48,
               p48, outb, sem, sem2):
    b = lax.axis_index("s") * 2 + lax.axis_index("c")

    lane = lax.broadcasted_iota(jnp.int32, (_L,), 0)
    inf_v = jnp.full((_L,), jnp.inf, jnp.float32)
    zero_v = jnp.full((_L,), _f32(0.0))
    big_v = jnp.full((_L,), _f32(_MASKED))

    n_groups = _N // _L

    pltpu.sync_copy(seg_hbm.at[pl.ds((b * 2 + 0) * _N, _N)], xb)
    pltpu.sync_copy(seg_hbm.at[pl.ds((b * 2 + 1) * _N, _N)], yb)

    def mask_body(i, carry):
        for j in range(8):
            sl = pl.ds((i * 8 + j) * _L, _L)
            mk[sl] = jnp.where(yb[sl] > xb[sl], zero_v, big_v)
        return carry

    lax.fori_loop(0, n_groups // 8, mask_body, jnp.int32(0))

    bsets = [(xb, yb, zb), (xb2, yb2, zb2)]
    bsems = [sem, sem2]

    def start_planes(k, bufs, dsem):
        if k < _NK:
            pb = (b * 24 + 3 * k) * _N
            src = kpts_hbm
        else:
            pb = (b * 3) * _N
            src = cpt_hbm
        return [pltpu.async_copy(src.at[pl.ds(pb + c * _N, _N)], bufs[c],
                                 dsem) for c in range(3)]

    pending = {0: start_planes(0, bsets[0], bsems[0]), 1: None}

    for k in range(_K):
        s = k % 2
        for h in pending[s]:
            h.wait()
        if k + 1 < _K:
            pending[1 - s] = start_planes(k + 1, bsets[1 - s], bsems[1 - s])
        cxb, cyb, czb = bsets[s]

        best_v[pl.ds(k * _L, _L)] = inf_v
        best_i[pl.ds(k * _L, _L)] = lane * 0

        def do_merge(mn, g, thr, k=k):
            def merge(mn=mn, g=g, k=k):
                gidx = g * _L + lane
                sv, si = plsc.sort_key_val(mn, gidx)
                rv = lax.rev(sv, (0,))
                ri = lax.rev(si, (0,))
                bv = best_v[pl.ds(k * _L, _L)]
                bi = best_i[pl.ds(k * _L, _L)]
                keep_old = bv <= rv
                mv = jnp.where(keep_old, bv, rv)
                mi = jnp.where(keep_old, bi, ri)
                mv, mi = plsc.sort_key_val(mv, mi)
                best_v[pl.ds(k * _L, _L)] = mv
                best_i[pl.ds(k * _L, _L)] = mi
                return jnp.min(jnp.where(lane == _NCAND - 1, mv, inf_v))

            def keep():
                return thr

            return lax.cond(jnp.min(mn) < thr, merge, keep)

        def scan_body(i, thr, k=k, cxb=cxb, cyb=cyb, czb=czb):
            g0 = i * _U
            mns = []
            for j in range(_U):
                sl = pl.ds((i * _U + j) * _L, _L)
                x = cxb[sl]
                y = cyb[sl]
                z = czb[sl]
                mns.append(x * x + y * y + z * z + mk[sl])
            t = mns
            while len(t) > 1:
                t = [jnp.minimum(t[2 * a], t[2 * a + 1])
                     for a in range(len(t) // 2)]
            blockmin = jnp.min(t[0])

            def hit(thr=thr, mns=mns, g0=g0, k=k):
                tc = thr
                for j in range(_U):
                    tc = do_merge(mns[j], g0 + j, tc, k=k)
                return tc

            def miss():
                return thr

            return lax.cond(blockmin < thr, hit, miss)

        lax.fori_loop(0, n_groups // _U, scan_body, _f32(jnp.inf))

    imax = jnp.full((_L,), jnp.int32(2147483647), jnp.int32)
    for k in range(_K):
        bv = best_v[pl.ds(k * _L, _L)]
        bi = best_i[pl.ds(k * _L, _L)]
        done = lane < 0
        for _ in range(_NCAND):
            cur = jnp.where(done, inf_v, bv)
            m = jnp.min(cur)
            candm = cur == jnp.full((_L,), _f32(1.0)) * m
            mi = jnp.min(jnp.where(candm, bi, imax))
            done = done | (candm & (bi == mi))

        if k < _NK:
            gbase = (b * 24 + 3 * k) * _N
            gsrc = kpts_hbm
        else:
            gbase = (b * 3) * _N
            gsrc = cpt_hbm
        idx48[pl.ds(0, _L)] = gbase + bi
        idx48[pl.ds(_L, _L)] = gbase + _N + bi
        idx48[pl.ds(2 * _L, _L)] = gbase + 2 * _N + bi
        pltpu.async_copy(gsrc.at[idx48], g48, sem).wait()
        pbase = (b * 3) * _N
        idx48[pl.ds(0, _L)] = pbase + bi
        idx48[pl.ds(_L, _L)] = pbase + _N + bi
        idx48[pl.ds(2 * _L, _L)] = pbase + 2 * _N + bi
        pltpu.async_copy(pcld_hbm.at[idx48], p48, sem).wait()
        cx = p48[pl.ds(0, _L)] + g48[pl.ds(0, _L)]
        cy = p48[pl.ds(_L, _L)] + g48[pl.ds(_L, _L)]
        cz = p48[pl.ds(2 * _L, _L)] + g48[pl.ds(2 * _L, _L)]

        w10 = jnp.where(done, jnp.full((_L,), _f32(1.0)), zero_v)
        inv10 = _f32(0.1)
        mx = jnp.sum(w10 * cx) * inv10
        my = jnp.sum(w10 * cy) * inv10
        mz = jnp.sum(w10 * cz) * inv10
        dx = cx - mx
        dy = cy - my
        dz = cz - mz
        vx = jnp.sum(w10 * dx * dx) * inv10
        vy = jnp.sum(w10 * dy * dy) * inv10
        vz = jnp.sum(w10 * dz * dz) * inv10

        def _thr2(var):
            varv = jnp.full((_L,), _f32(1.0)) * var
            i = lax.bitcast_convert_type(jnp.maximum(varv, _f32(1e-30)),
                                         jnp.int32)
            s_a = lax.bitcast_convert_type(
                (i >> 1) + jnp.int32(0x1FBD1DF6), jnp.float32)
            s_a = _f32(0.5) * (s_a + varv / s_a)
            s_a = _f32(0.5) * (s_a + varv / s_a)
            s_a = jnp.where(varv > zero_v, s_a, zero_v)
            return varv + _f32(2e-9) * s_a + _f32(1e-18)

        inl = ((dx * dx <= _thr2(vx))
               & (dy * dy <= _thr2(vy))
               & (dz * dz <= _thr2(vz))
               & done)
        ones = jnp.full((_L,), _f32(1.0))
        w = jnp.where(inl, ones, zero_v)
        denv = ones * (jnp.sum(w) + _f32(1e-8))
        oxv = (ones * jnp.sum(w * cx)) / denv
        oyv = (ones * jnp.sum(w * cy)) / denv
        ozv = (ones * jnp.sum(w * cz)) / denv

        outv = jnp.where(lane == 0, oxv,
                         jnp.where(lane == 1, oyv,
                                   jnp.where(lane == 2, ozv, zero_v)))
        outb[pl.ds(k * _L, _L)] = outv

    pltpu.sync_copy(outb, out_hbm.at[pl.ds(b * _K * _L, _K * _L)])


def kernel(pcld_input, kpts_pre_input, cpt_pre_input, seg_pre_input):
    b, n, nk, _ = kpts_pre_input.shape
    assert (b, n, nk) == (_B, _N, _NK)
    kpts_t = jnp.transpose(kpts_pre_input.reshape(b, n, 3 * _NK),
                           (0, 2, 1)).reshape(-1)
    cpt_t = jnp.transpose(cpt_pre_input.reshape(b, n, 3), (0, 2, 1)
                          ).reshape(-1)
    pcld_t = jnp.transpose(pcld_input, (0, 2, 1)).reshape(-1)
    seg_t = jnp.transpose(seg_pre_input, (0, 2, 1)).reshape(-1)

    mesh = plsc.VectorSubcoreMesh(core_axis_name="c", subcore_axis_name="s")
    run = functools.partial(
        pl.kernel,
        mesh=mesh,
        compiler_params=pltpu.CompilerParams(needs_layout_passes=False),
        out_type=jax.ShapeDtypeStruct((b * _K * _L,), jnp.float32),
        scratch_types=[
            pltpu.VMEM((_N,), jnp.float32),
            pltpu.VMEM((_N,), jnp.float32),
            pltpu.VMEM((_N,), jnp.float32),
            pltpu.VMEM((_N,), jnp.float32),
            pltpu.VMEM((_N,), jnp.float32),
            pltpu.VMEM((_N,), jnp.float32),
            pltpu.VMEM((_N,), jnp.float32),
            pltpu.VMEM((_K * _L,), jnp.float32),
            pltpu.VMEM((_K * _L,), jnp.int32),
            pltpu.VMEM((3 * _L,), jnp.int32),
            pltpu.VMEM((3 * _L,), jnp.float32),
            pltpu.VMEM((3 * _L,), jnp.float32),
            pltpu.VMEM((_K * _L,), jnp.float32),
            pltpu.SemaphoreType.DMA,
            pltpu.SemaphoreType.DMA,
        ],
    )(_sc_kernel)
    out16 = run(kpts_t, cpt_t, pcld_t, seg_t)
    return out16.reshape(b, _K, _L)[:, :, :3]
